# baseline (speedup 1.0000x reference)
import functools

import jax
import jax.numpy as jnp
from jax import lax
from jax.experimental import pallas as pl
from jax.experimental.pallas import tpu as pltpu
from jax.experimental.pallas import tpu_sc as plsc

NUM_CORES = 2
NUM_SUBCORES = 16
NW = NUM_CORES * NUM_SUBCORES
B = 16384
D = 32
BPW = B // NW
CH = 128
NCH = BPW // CH


def _gather_body(utab, uidx, itab, iidx, uout, iout, uidx_v, iidx_v,
                 urows_v, irows_v, sem):
    wid = lax.axis_index("s") * NUM_CORES + lax.axis_index("c")
    base = wid * BPW
    pltpu.sync_copy(uidx.at[wid], uidx_v)
    pltpu.sync_copy(iidx.at[wid], iidx_v)
    descs = []
    for j in range(NCH):
        descs.append(pltpu.async_copy(
            utab.at[uidx_v.at[j]], urows_v.at[pl.ds(j * CH, CH)], sem))
        descs.append(pltpu.async_copy(
            itab.at[iidx_v.at[j]], irows_v.at[pl.ds(j * CH, CH)], sem))
    for d in descs:
        d.wait()
    pltpu.sync_copy(urows_v, uout.at[pl.ds(base, BPW)])
    pltpu.sync_copy(irows_v, iout.at[pl.ds(base, BPW)])


_gather = pl.kernel(
    _gather_body,
    mesh=plsc.VectorSubcoreMesh(core_axis_name="c", subcore_axis_name="s"),
    out_type=[
        jax.ShapeDtypeStruct((B, D), jnp.float32),
        jax.ShapeDtypeStruct((B, D), jnp.float32),
    ],
    scratch_types=[
        pltpu.VMEM((NCH, CH), jnp.int32),
        pltpu.VMEM((NCH, CH), jnp.int32),
        pltpu.VMEM((BPW, D), jnp.float32),
        pltpu.VMEM((BPW, D), jnp.float32),
        pltpu.SemaphoreType.DMA,
    ],
    compiler_params=pltpu.CompilerParams(use_tc_tiling_on_sc=False),
)


BLK = 2048


def _mlp_body(uv, iv, w1u, w1i, b1, w2t, b2, w3, b3, out):
    h = jnp.dot(uv[...], w1u[...], preferred_element_type=jnp.float32)
    h = h + jnp.dot(iv[...], w1i[...], preferred_element_type=jnp.float32)
    h = jnp.maximum(h + b1[...], 0.0)
    h2 = jnp.dot(h, w2t[...], preferred_element_type=jnp.float32) + b2[...]
    h2 = jnp.maximum(h2, 0.0)
    out[...] = jnp.sum(h2 * w3[...], axis=1) + b3[0, 0]


def _mlp(uv, iv, w1u, w1i, b1, w2t, b2, w3, b3):
    grid = (B // BLK,)
    return pl.pallas_call(
        _mlp_body,
        grid=grid,
        in_specs=[
            pl.BlockSpec((BLK, D), lambda i: (i, 0)),
            pl.BlockSpec((BLK, D), lambda i: (i, 0)),
            pl.BlockSpec((D, 64), lambda i: (0, 0)),
            pl.BlockSpec((D, 64), lambda i: (0, 0)),
            pl.BlockSpec((1, 64), lambda i: (0, 0)),
            pl.BlockSpec((64, 32), lambda i: (0, 0)),
            pl.BlockSpec((1, 32), lambda i: (0, 0)),
            pl.BlockSpec((1, 32), lambda i: (0, 0)),
            pl.BlockSpec((1, 1), lambda i: (0, 0), memory_space=pltpu.SMEM),
        ],
        out_specs=pl.BlockSpec((BLK,), lambda i: (i,)),
        out_shape=jax.ShapeDtypeStruct((B,), jnp.float32),
    )(uv, iv, w1u, w1i, b1, w2t, b2, w3, b3)


def kernel(user_indices, item_indices, user_table, item_table,
           W1, b1, W2, b2, W3, b3):
    uidx = user_indices.astype(jnp.int32).reshape(NW, NCH, CH)
    iidx = item_indices.astype(jnp.int32).reshape(NW, NCH, CH)
    uv, iv = _gather(user_table, uidx, item_table, iidx)
    w1u = W1[:, :D].T
    w1i = W1[:, D:].T
    return _mlp(uv, iv, w1u, w1i, b1.reshape(1, 64), W2.T,
                b2.reshape(1, 32), W3, b3.reshape(1, 1))
